# baseline (speedup 1.0000x reference)
import functools

import jax
import jax.numpy as jnp
from jax import lax
from jax.experimental import pallas as pl
from jax.experimental.pallas import tpu as pltpu
from jax.experimental.pallas import tpu_sc as plsc

_PCT = 95
_MIN_SCALE = 1e-06
_MAX_SCALE = 1000000.0
_INF_BITS = 0x7F800000

_N = 128 * 8192
_NS = 16
_L = 16
_PER_T = _N // _NS
_VECS = _PER_T // _L


def _tc1_body(x_ref, p_ref, s_ref):
    x = x_ref[:]
    a = jnp.abs(x)
    mask = a > 1e-08
    n0 = jnp.sum(mask.astype(jnp.int32))
    n0f = n0.astype(jnp.float32)
    s = jnp.sum(jnp.where(mask, a, 0.0))
    mean = s / jnp.maximum(n0f, 1.0)
    d = a - mean
    ss = jnp.sum(jnp.where(mask, d * d, 0.0))
    var = ss / jnp.maximum(n0f - 1.0, 1.0)
    std = jnp.sqrt(var)
    refined = mask & (jnp.abs(d) <= 3.0 * std)
    nr = jnp.sum(refined.astype(jnp.int32))
    use_refined = (n0 > 10) & (nr > 0)
    n = jnp.where(use_refined, nr, n0)
    k = jnp.clip((_PCT * n) // 100, 0, n - 1)
    r = k + 1
    sel = (refined & use_refined) | (mask & jnp.logical_not(use_refined))
    bits = lax.bitcast_convert_type(a, jnp.int32)
    p_ref[:] = jnp.where(sel, bits, _INF_BITS)
    rows = lax.broadcasted_iota(jnp.int32, (8, 128), 0)
    s_ref[:] = jnp.where(
        rows == 0, r, jnp.where(rows == 1, n, jnp.where(rows == 2, n0, 0))
    )


def _tc2_body(x_ref, d_ref, o_ref):
    o_ref[:] = x_ref[:] / d_ref[0, 0]


def _cum_search(ghist_v, tmpa_v, tmpb_v, r_spl, nb):
    def chunk(j, carry):
        tot, bfound, cbel = carry
        h = ghist_v[pl.ds(j * _L, _L)]
        cs = plsc.cumsum(h) + tot
        ge = cs >= r_spl
        anyv = plsc.all_reduce_population_count(ge)
        ffs = plsc.all_reduce_ffs(ge)
        ffs = jnp.minimum(ffs, _L - 1)
        excl = cs - h
        tmpa_v[...] = excl
        gathered = plsc.load_gather(tmpa_v, [ffs])
        tmpb_v[...] = cs
        tot_new = plsc.load_gather(tmpb_v, [jnp.full((_L,), _L - 1, jnp.int32)])
        newly = (bfound < 0) & (anyv > 0)
        bfound = jnp.where(newly, j * _L + ffs, bfound)
        cbel = jnp.where(newly, gathered, cbel)
        return (tot_new, bfound, cbel)

    zero = jnp.zeros((_L,), jnp.int32)
    init = (zero, zero - 1, zero)
    tot, bfound, cbel = lax.fori_loop(0, nb // _L, chunk, init)
    return jnp.maximum(bfound, 0), cbel


_HSTRIDE = 2064
_DUMMY_BIN = 2048


def _zero_hist(h_ref, nb):
    zero = jnp.zeros((_L,), jnp.int32)
    for row in range(_NS):

        @plsc.parallel_loop(0, nb // _L, unroll=8)
        def _(col, row=row):
            h_ref[pl.ds(row * _HSTRIDE + col * _L, _L)] = zero


def _hist_round(p_v, h_ref, rowbuf_v, sh_ref, ghist_v, tmpa_v, tmpb_v, sid,
                r_spl, nb, bin_fn, mask_fn):
    _zero_hist(h_ref, nb)
    lane_off = lax.broadcasted_iota(jnp.int32, (_L,), 0) * _HSTRIDE
    ones = jnp.ones((_L,), jnp.int32)

    @plsc.parallel_loop(0, _VECS, unroll=8)
    def _(i):
        v = p_v[pl.ds(i * _L, _L)]
        bins = jnp.where(mask_fn(v), bin_fn(v), _DUMMY_BIN)
        plsc.addupdate_scatter(h_ref, [lane_off + bins], ones)

    @plsc.parallel_loop(0, nb // _L, unroll=4)
    def _(j):
        acc = jnp.zeros((_L,), jnp.int32)
        for row in range(_NS):
            acc = acc + h_ref[pl.ds(row * _HSTRIDE + j * _L, _L)]
        rowbuf_v[pl.ds(j * _L, _L)] = acc

    pltpu.sync_copy(rowbuf_v.at[pl.ds(0, nb)], sh_ref.at[pl.ds(sid * nb, nb)])
    plsc.subcore_barrier()
    for row in range(_NS):
        pltpu.sync_copy(
            sh_ref.at[pl.ds(row * nb, nb)], h_ref.at[pl.ds(row * _HSTRIDE, nb)]
        )

    @plsc.parallel_loop(0, nb // _L, unroll=4)
    def _(j):
        acc = jnp.zeros((_L,), jnp.int32)
        for row in range(_NS):
            acc = acc + h_ref[pl.ds(row * _HSTRIDE + j * _L, _L)]
        ghist_v[pl.ds(j * _L, _L)] = acc

    return _cum_search(ghist_v, tmpa_v, tmpb_v, r_spl, nb)


def _sc_select_make():
    mesh = plsc.VectorSubcoreMesh(
        core_axis_name="c", subcore_axis_name="s", num_cores=2, num_subcores=_NS
    )

    @functools.partial(
        pl.kernel,
        out_type=jax.ShapeDtypeStruct((_L,), jnp.int32),
        mesh=mesh,
        compiler_params=pltpu.CompilerParams(needs_layout_passes=False),
        scratch_types=dict(
            p_v=pltpu.VMEM((_PER_T,), jnp.int32),
            h_v=pltpu.VMEM((_NS * _HSTRIDE,), jnp.int32),
            rowbuf_v=pltpu.VMEM((2048,), jnp.int32),
            ghist_v=pltpu.VMEM((2048,), jnp.int32),
            r_v=pltpu.VMEM((_L,), jnp.int32),
            tmpa_v=pltpu.VMEM((_L,), jnp.int32),
            tmpb_v=pltpu.VMEM((_L,), jnp.int32),
            out_v=pltpu.VMEM((_L,), jnp.int32),
            sh_a=pltpu.VMEM_SHARED((_NS * 1024,), jnp.int32),
            sh_b=pltpu.VMEM_SHARED((_NS * 2048,), jnp.int32),
            sh_c=pltpu.VMEM_SHARED((_NS * 1024,), jnp.int32),
        ),
    )
    def sc_select(p_hbm, r_hbm, ans_hbm, *, p_v, h_v, rowbuf_v, ghist_v, r_v,
                  tmpa_v, tmpb_v, out_v, sh_a, sh_b, sh_c):
        cid = lax.axis_index("c")
        sid = lax.axis_index("s")
        pltpu.sync_copy(p_hbm.at[pl.ds(sid * _PER_T, _PER_T)], p_v)
        pltpu.sync_copy(r_hbm, r_v)
        r1 = r_v[...]

        b1, cb1 = _hist_round(
            p_v, h_v, rowbuf_v, sh_a, ghist_v, tmpa_v, tmpb_v, sid, r1, 1024,
            lambda v: lax.shift_right_logical(v, 21),
            lambda v: jnp.ones((_L,), jnp.bool_),
        )
        r2 = r1 - cb1

        b2, cb2 = _hist_round(
            p_v, h_v, rowbuf_v, sh_b, ghist_v, tmpa_v, tmpb_v, sid, r2, 2048,
            lambda v: lax.shift_right_logical(v, 10) & 0x7FF,
            lambda v: lax.shift_right_logical(v, 21) == b1,
        )
        r3 = r2 - cb2
        pre2 = (b1 << 11) | b2

        b3, _ = _hist_round(
            p_v, h_v, rowbuf_v, sh_c, ghist_v, tmpa_v, tmpb_v, sid, r3, 1024,
            lambda v: v & 0x3FF,
            lambda v: lax.shift_right_logical(v, 10) == pre2,
        )

        ans = (b1 << 21) | (b2 << 10) | b3

        @pl.when((cid == 0) & (sid == 0))
        def _():
            out_v[...] = ans
            pltpu.sync_copy(out_v, ans_hbm)

    return sc_select


def kernel(x):
    p, stats = pl.pallas_call(
        _tc1_body,
        out_shape=(
            jax.ShapeDtypeStruct(x.shape, jnp.int32),
            jax.ShapeDtypeStruct((8, 128), jnp.int32),
        ),
    )(x)
    r = stats[0, 0]
    n = stats[1, 0]
    n0 = stats[2, 0]
    rvec = jnp.broadcast_to(r, (_L,))
    ansv = _sc_select_make()(p.reshape(-1), rvec)
    ans = ansv[0]
    val = lax.bitcast_convert_type(ans, jnp.float32)
    val = jnp.where(n == 0, 1.0, val)
    value = jnp.clip(val, _MIN_SCALE, _MAX_SCALE)
    value = jnp.where(n0 == 0, 1.0, value)
    value = jnp.clip(value, _MIN_SCALE, _MAX_SCALE)
    denom = (value + 1e-08).reshape(1, 1)
    return pl.pallas_call(
        _tc2_body,
        out_shape=jax.ShapeDtypeStruct(x.shape, x.dtype),
        in_specs=[
            pl.BlockSpec(memory_space=pltpu.VMEM),
            pl.BlockSpec(memory_space=pltpu.SMEM),
        ],
        out_specs=pl.BlockSpec(memory_space=pltpu.VMEM),
    )(x, denom)


# speedup vs baseline: 1.3917x; 1.3917x over previous
import functools

import jax
import jax.numpy as jnp
from jax import lax
from jax.experimental import pallas as pl
from jax.experimental.pallas import tpu as pltpu
from jax.experimental.pallas import tpu_sc as plsc

_PCT = 95
_MIN_SCALE = 1e-06
_MAX_SCALE = 1000000.0
_INF_BITS = 0x7F800000

_N = 128 * 8192
_NS = 16
_L = 16
_PER_T = _N // _NS
_VECS = _PER_T // _L


def _tc1_body(x_ref, p_ref, s_ref):
    x = x_ref[:]
    a = jnp.abs(x)
    mask = a > 1e-08
    n0 = jnp.sum(mask.astype(jnp.int32))
    n0f = n0.astype(jnp.float32)
    s = jnp.sum(jnp.where(mask, a, 0.0))
    mean = s / jnp.maximum(n0f, 1.0)
    d = a - mean
    ss = jnp.sum(jnp.where(mask, d * d, 0.0))
    var = ss / jnp.maximum(n0f - 1.0, 1.0)
    std = jnp.sqrt(var)
    refined = mask & (jnp.abs(d) <= 3.0 * std)
    nr = jnp.sum(refined.astype(jnp.int32))
    use_refined = (n0 > 10) & (nr > 0)
    n = jnp.where(use_refined, nr, n0)
    k = jnp.clip((_PCT * n) // 100, 0, n - 1)
    r = k + 1
    sel = (refined & use_refined) | (mask & jnp.logical_not(use_refined))
    bits = lax.bitcast_convert_type(a, jnp.int32)
    p_ref[:] = jnp.where(sel, bits, _INF_BITS)
    rows = lax.broadcasted_iota(jnp.int32, (8, 128), 0)
    s_ref[:] = jnp.where(
        rows == 0, r, jnp.where(rows == 1, n, jnp.where(rows == 2, n0, 0))
    )


def _tc2_body(x_ref, d_ref, o_ref):
    o_ref[:] = x_ref[:] / d_ref[0, 0]


def _cum_search(ghist_v, tmpa_v, tmpb_v, r_spl, nb):
    def chunk(j, carry):
        tot, bfound, cbel = carry
        h = ghist_v[pl.ds(j * _L, _L)]
        cs = plsc.cumsum(h) + tot
        ge = cs >= r_spl
        anyv = plsc.all_reduce_population_count(ge)
        ffs = plsc.all_reduce_ffs(ge)
        ffs = jnp.minimum(ffs, _L - 1)
        excl = cs - h
        tmpa_v[...] = excl
        gathered = plsc.load_gather(tmpa_v, [ffs])
        tmpb_v[...] = cs
        tot_new = plsc.load_gather(tmpb_v, [jnp.full((_L,), _L - 1, jnp.int32)])
        newly = (bfound < 0) & (anyv > 0)
        bfound = jnp.where(newly, j * _L + ffs, bfound)
        cbel = jnp.where(newly, gathered, cbel)
        return (tot_new, bfound, cbel)

    zero = jnp.zeros((_L,), jnp.int32)
    init = (zero, zero - 1, zero)
    tot, bfound, cbel = lax.fori_loop(0, nb // _L, chunk, init)
    return jnp.maximum(bfound, 0), cbel


_SKEW = 2081
_ROW = 2080
_DUMMY = 2064


def _zero_hist(h_ref, nb):
    zero = jnp.zeros((_L,), jnp.int32)
    for row in range(_NS):

        @plsc.parallel_loop(0, nb // _L, unroll=8)
        def _(col, row=row):
            h_ref[pl.ds(row * _SKEW + col * _L, _L)] = zero


def _hist_round(p_v, h_ref, rowbuf_v, sh_ref, ghist_v, tmpa_v, tmpb_v, sid,
                r_spl, nb, bin_fn, mask_fn):
    _zero_hist(h_ref, nb)
    lane_off = lax.broadcasted_iota(jnp.int32, (_L,), 0) * _SKEW
    ones = jnp.ones((_L,), jnp.int32)

    @plsc.parallel_loop(0, _VECS, unroll=16)
    def _(i):
        v = p_v[pl.ds(i * _L, _L)]
        bins = jnp.where(mask_fn(v), bin_fn(v), _DUMMY)
        plsc.addupdate_scatter(h_ref, [lane_off + bins], ones)

    @plsc.parallel_loop(0, nb // _L, unroll=4)
    def _(j):
        acc = jnp.zeros((_L,), jnp.int32)
        for row in range(_NS):
            acc = acc + h_ref[pl.ds(row * _SKEW + j * _L, _L)]
        rowbuf_v[pl.ds(j * _L, _L)] = acc

    pltpu.sync_copy(rowbuf_v.at[pl.ds(0, nb)], sh_ref.at[pl.ds(sid * nb, nb)])
    plsc.subcore_barrier()
    for row in range(_NS):
        pltpu.sync_copy(
            sh_ref.at[pl.ds(row * nb, nb)], h_ref.at[pl.ds(row * _ROW, nb)]
        )

    @plsc.parallel_loop(0, nb // _L, unroll=4)
    def _(j):
        acc = jnp.zeros((_L,), jnp.int32)
        for row in range(_NS):
            acc = acc + h_ref[pl.ds(row * _ROW + j * _L, _L)]
        ghist_v[pl.ds(j * _L, _L)] = acc

    return _cum_search(ghist_v, tmpa_v, tmpb_v, r_spl, nb)


def _sc_select_make():
    mesh = plsc.VectorSubcoreMesh(
        core_axis_name="c", subcore_axis_name="s", num_cores=2, num_subcores=_NS
    )

    @functools.partial(
        pl.kernel,
        out_type=jax.ShapeDtypeStruct((_L,), jnp.int32),
        mesh=mesh,
        compiler_params=pltpu.CompilerParams(needs_layout_passes=False),
        scratch_types=dict(
            p_v=pltpu.VMEM((_PER_T,), jnp.int32),
            h_v=pltpu.VMEM((_NS * _ROW,), jnp.int32),
            rowbuf_v=pltpu.VMEM((2048,), jnp.int32),
            ghist_v=pltpu.VMEM((2048,), jnp.int32),
            r_v=pltpu.VMEM((_L,), jnp.int32),
            tmpa_v=pltpu.VMEM((_L,), jnp.int32),
            tmpb_v=pltpu.VMEM((_L,), jnp.int32),
            out_v=pltpu.VMEM((_L,), jnp.int32),
            sh_a=pltpu.VMEM_SHARED((_NS * 1024,), jnp.int32),
            sh_b=pltpu.VMEM_SHARED((_NS * 2048,), jnp.int32),
            sh_c=pltpu.VMEM_SHARED((_NS * 1024,), jnp.int32),
        ),
    )
    def sc_select(p_hbm, r_hbm, ans_hbm, *, p_v, h_v, rowbuf_v, ghist_v, r_v,
                  tmpa_v, tmpb_v, out_v, sh_a, sh_b, sh_c):
        cid = lax.axis_index("c")
        sid = lax.axis_index("s")
        pltpu.sync_copy(p_hbm.at[pl.ds(sid * _PER_T, _PER_T)], p_v)
        pltpu.sync_copy(r_hbm, r_v)
        r1 = r_v[...]

        b1, cb1 = _hist_round(
            p_v, h_v, rowbuf_v, sh_a, ghist_v, tmpa_v, tmpb_v, sid, r1, 1024,
            lambda v: lax.shift_right_logical(v, 21),
            lambda v: jnp.ones((_L,), jnp.bool_),
        )
        r2 = r1 - cb1

        b2, cb2 = _hist_round(
            p_v, h_v, rowbuf_v, sh_b, ghist_v, tmpa_v, tmpb_v, sid, r2, 2048,
            lambda v: lax.shift_right_logical(v, 10) & 0x7FF,
            lambda v: lax.shift_right_logical(v, 21) == b1,
        )
        r3 = r2 - cb2
        pre2 = (b1 << 11) | b2

        b3, _ = _hist_round(
            p_v, h_v, rowbuf_v, sh_c, ghist_v, tmpa_v, tmpb_v, sid, r3, 1024,
            lambda v: v & 0x3FF,
            lambda v: lax.shift_right_logical(v, 10) == pre2,
        )

        ans = (b1 << 21) | (b2 << 10) | b3

        @pl.when((cid == 0) & (sid == 0))
        def _():
            out_v[...] = ans
            pltpu.sync_copy(out_v, ans_hbm)

    return sc_select


def kernel(x):
    p, stats = pl.pallas_call(
        _tc1_body,
        out_shape=(
            jax.ShapeDtypeStruct(x.shape, jnp.int32),
            jax.ShapeDtypeStruct((8, 128), jnp.int32),
        ),
    )(x)
    r = stats[0, 0]
    n = stats[1, 0]
    n0 = stats[2, 0]
    rvec = jnp.broadcast_to(r, (_L,))
    ansv = _sc_select_make()(p.reshape(-1), rvec)
    ans = ansv[0]
    val = lax.bitcast_convert_type(ans, jnp.float32)
    val = jnp.where(n == 0, 1.0, val)
    value = jnp.clip(val, _MIN_SCALE, _MAX_SCALE)
    value = jnp.where(n0 == 0, 1.0, value)
    value = jnp.clip(value, _MIN_SCALE, _MAX_SCALE)
    denom = (value + 1e-08).reshape(1, 1)
    return pl.pallas_call(
        _tc2_body,
        out_shape=jax.ShapeDtypeStruct(x.shape, x.dtype),
        in_specs=[
            pl.BlockSpec(memory_space=pltpu.VMEM),
            pl.BlockSpec(memory_space=pltpu.SMEM),
        ],
        out_specs=pl.BlockSpec(memory_space=pltpu.VMEM),
    )(x, denom)
